# Initial kernel scaffold; baseline (speedup 1.0000x reference)
import jax
import jax.numpy as jnp
from jax.experimental import pallas as pl


def kernel(node_attr, edge_index, edge_attr, u, batch, W1, b1, W2, b2):
    raise NotImplementedError("write your pallas kernel here")


# speedup vs baseline: 2.1829x; 2.1829x over previous
import functools

import jax
import jax.numpy as jnp
from jax import lax
from jax.experimental import pallas as pl
from jax.experimental.pallas import tpu as pltpu
from jax.experimental.pallas import tpu_sc as plsc

N = 10000
E = 320000
D = 128
B = 64
H1 = 256
H2 = 128

NC = 2
NS = 16
EPT = E // NS
C = 80
NCH = EPT // C
NH = 5120
TRASH = NH
NHP = NH + 8
RPT = NH // NS
ZR = 160

_mesh = plsc.VectorSubcoreMesh(core_axis_name="c", subcore_axis_name="s")


def _remap_indices(idx_v, c):
    lo = c * NH

    def remap(kk, _):
        for i in range(C // 16):
            iv = idx_v[kk, pl.ds(i * 16, 16)]
            local = iv - lo
            inrange = (local >= 0) & (local < NH)
            idx_v[kk, pl.ds(i * 16, 16)] = jnp.where(inrange, local, TRASH)
        return 0

    lax.fori_loop(0, NCH, remap, 0)


@functools.partial(
    pl.kernel,
    mesh=_mesh,
    out_type=[
        jax.ShapeDtypeStruct((NC, NH, D), jnp.float32),
    ],
    scratch_types=[
        pltpu.VMEM((NCH, C), jnp.int32),
        pltpu.VMEM((C, D), jnp.float32),
        pltpu.VMEM((ZR, D), jnp.float32),
        pltpu.VMEM_SHARED((NHP, D), jnp.float32),
    ],
)
def _sc_scatter(edge_hbm, col3_hbm, sums_hbm, idx_v, rows_v, zero_v, acc_sh):
    t = lax.axis_index("s")
    c = lax.axis_index("c")

    zeros16 = jnp.zeros((16,), jnp.float32)

    def zrow(i, _):
        for j in range(D // 16):
            zero_v[i, pl.ds(j * 16, 16)] = zeros16
        return 0

    lax.fori_loop(0, ZR, zrow, 0)

    for j in range(RPT // ZR):
        pltpu.sync_copy(zero_v, acc_sh.at[pl.ds(t * RPT + j * ZR, ZR)])

    @pl.when(t == 0)
    def _zero_trash():
        pltpu.sync_copy(zero_v.at[pl.ds(0, 8)], acc_sh.at[pl.ds(NH, 8)])

    plsc.subcore_barrier()

    pltpu.sync_copy(col3_hbm.at[t], idx_v)
    _remap_indices(idx_v, c)

    def body(k, _):
        base = t * EPT + k * C
        pltpu.sync_copy(edge_hbm.at[pl.ds(base, C)], rows_v)
        pltpu.sync_copy(rows_v, acc_sh.at[idx_v.at[k]], add=True)
        return 0

    lax.fori_loop(0, NCH, body, 0)

    plsc.subcore_barrier()

    pltpu.sync_copy(acc_sh.at[pl.ds(t * RPT, RPT)],
                    sums_hbm.at[c, pl.ds(t * RPT, RPT)])


@functools.partial(
    pl.kernel,
    mesh=_mesh,
    out_type=[
        jax.ShapeDtypeStruct((NC, NH, 16), jnp.float32),
    ],
    scratch_types=[
        pltpu.VMEM((NCH, C), jnp.int32),
        pltpu.VMEM((C, 16), jnp.float32),
        pltpu.VMEM((ZR, 16), jnp.float32),
        pltpu.VMEM_SHARED((NHP, 16), jnp.float32),
    ],
)
def _sc_count(col3_hbm, counts_hbm, idx_v, ones_v, zero16_v, cnt_sh):
    t = lax.axis_index("s")
    c = lax.axis_index("c")

    zeros16 = jnp.zeros((16,), jnp.float32)
    ones16 = jnp.ones((16,), jnp.float32)

    def zrow(i, _):
        zero16_v[i, :] = zeros16
        return 0

    lax.fori_loop(0, ZR, zrow, 0)

    def orow(i, _):
        ones_v[i, :] = ones16
        return 0

    lax.fori_loop(0, C, orow, 0)

    for j in range(RPT // ZR):
        pltpu.sync_copy(zero16_v, cnt_sh.at[pl.ds(t * RPT + j * ZR, ZR)])

    @pl.when(t == 0)
    def _zero_trash():
        pltpu.sync_copy(zero16_v.at[pl.ds(0, 8)], cnt_sh.at[pl.ds(NH, 8)])

    plsc.subcore_barrier()

    pltpu.sync_copy(col3_hbm.at[t], idx_v)
    _remap_indices(idx_v, c)

    def body(k, _):
        pltpu.sync_copy(ones_v, cnt_sh.at[idx_v.at[k]], add=True)
        return 0

    lax.fori_loop(0, NCH, body, 0)

    plsc.subcore_barrier()

    pltpu.sync_copy(cnt_sh.at[pl.ds(t * RPT, RPT)],
                    counts_hbm.at[c, pl.ds(t * RPT, RPT)])


BN = 1000
_GRID = N // BN


def _mlp_body(sums_ref, counts_ref, node_ref, batch_ref, u_ref,
              w1_ref, b1_ref, w2_ref, b2_ref, out_ref):
    s = sums_ref[...]
    cnt = counts_ref[:, 0]
    agg = s / jnp.maximum(cnt, 1.0)[:, None]
    bvec = batch_ref[0, 0, :]
    onehot = (bvec[:, None]
              == lax.broadcasted_iota(jnp.int32, (BN, B), 1)
              ).astype(jnp.float32)
    ub = jnp.dot(onehot, u_ref[...],
                 preferred_element_type=jnp.float32)
    x = jnp.concatenate([node_ref[...], agg, ub], axis=1)
    h = jnp.dot(x, w1_ref[...], preferred_element_type=jnp.float32,
                precision=lax.Precision.HIGHEST) + b1_ref[...]
    h = jnp.maximum(h, 0.0)
    y = jnp.dot(h, w2_ref[...], preferred_element_type=jnp.float32,
                precision=lax.Precision.HIGHEST) + b2_ref[...]
    out_ref[...] = y


_mlp_call = pl.pallas_call(
    _mlp_body,
    grid=(_GRID,),
    in_specs=[
        pl.BlockSpec((BN, D), lambda i: (i, 0)),
        pl.BlockSpec((BN, 16), lambda i: (i, 0)),
        pl.BlockSpec((BN, D), lambda i: (i, 0)),
        pl.BlockSpec((1, 1, BN), lambda i: (i, 0, 0)),
        pl.BlockSpec((B, D), lambda i: (0, 0)),
        pl.BlockSpec((3 * D, H1), lambda i: (0, 0)),
        pl.BlockSpec((1, H1), lambda i: (0, 0)),
        pl.BlockSpec((H1, H2), lambda i: (0, 0)),
        pl.BlockSpec((1, H2), lambda i: (0, 0)),
    ],
    out_specs=pl.BlockSpec((BN, H2), lambda i: (i, 0)),
    out_shape=jax.ShapeDtypeStruct((N, H2), jnp.float32),
)


@jax.jit
def kernel(node_attr, edge_index, edge_attr, u, batch, W1, b1, W2, b2):
    col3 = edge_index[1].reshape(NS, NCH, C)
    (sums,) = _sc_scatter(edge_attr, col3)
    (counts,) = _sc_count(col3)
    sums_n = sums.reshape(NC * NH, D)
    counts_n = counts.reshape(NC * NH, 16)
    batch3 = batch.reshape(_GRID, 1, BN)
    return _mlp_call(sums_n, counts_n, node_attr, batch3, u,
                     W1, b1.reshape(1, H1), W2, b2.reshape(1, H2))


# speedup vs baseline: 3.2862x; 1.5054x over previous
import functools

import jax
import jax.numpy as jnp
from jax import lax
from jax.experimental import pallas as pl
from jax.experimental.pallas import tpu as pltpu
from jax.experimental.pallas import tpu_sc as plsc

N = 10000
E = 320000
D = 128
B = 64
H1 = 256
H2 = 128

NC = 2
NS = 16
EPT = E // NS
C = 80
NCH = EPT // C
CL = C
NL = EPT // CL
NH = 5120
TRASH = NH
NHP = NH + 8
RPT = NH // NS
ZR = 160

_mesh = plsc.VectorSubcoreMesh(core_axis_name="c", subcore_axis_name="s")


def _remap_indices(idx_v, c):
    lo = c * NH

    def remap(kk, _):
        for i in range(C // 16):
            iv = idx_v[kk, pl.ds(i * 16, 16)]
            local = iv - lo
            inrange = (local >= 0) & (local < NH)
            idx_v[kk, pl.ds(i * 16, 16)] = jnp.where(inrange, local, TRASH)
        return 0

    lax.fori_loop(0, NCH, remap, 0)


@functools.partial(
    pl.kernel,
    mesh=_mesh,
    out_type=[
        jax.ShapeDtypeStruct((NC, NH, D), jnp.float32),
    ],
    scratch_types=[
        pltpu.VMEM((NCH, C), jnp.int32),
        pltpu.VMEM((2, CL, D), jnp.float32),
        pltpu.VMEM((ZR, D), jnp.float32),
        pltpu.VMEM_SHARED((NHP, D), jnp.float32),
        pltpu.SemaphoreType.DMA,
        pltpu.SemaphoreType.DMA,
    ],
)
def _sc_scatter(edge_hbm, col3_hbm, sums_hbm, idx_v, rows_v, zero_v, acc_sh,
                sem0, sem1):
    t = lax.axis_index("s")
    c = lax.axis_index("c")

    zeros16 = jnp.zeros((16,), jnp.float32)

    def zrow(i, _):
        for j in range(D // 16):
            zero_v[i, pl.ds(j * 16, 16)] = zeros16
        return 0

    lax.fori_loop(0, ZR, zrow, 0)

    for j in range(RPT // ZR):
        pltpu.sync_copy(zero_v, acc_sh.at[pl.ds(t * RPT + j * ZR, ZR)])

    @pl.when(t == 0)
    def _zero_trash():
        pltpu.sync_copy(zero_v.at[pl.ds(0, 8)], acc_sh.at[pl.ds(NH, 8)])

    plsc.subcore_barrier()

    pltpu.sync_copy(col3_hbm.at[t], idx_v)
    _remap_indices(idx_v, c)

    ebase = t * EPT
    sems = (sem0, sem1)

    def load(l, buf):
        pltpu.async_copy(edge_hbm.at[pl.ds(ebase + l * CL, CL)],
                         rows_v.at[buf], sems[buf])

    def wait(buf):
        pltpu.make_async_copy(edge_hbm.at[pl.ds(0, CL)], rows_v.at[buf],
                              sems[buf]).wait()

    def scat(l, buf):
        pltpu.sync_copy(rows_v.at[buf], acc_sh.at[idx_v.at[l]], add=True)

    load(0, 0)
    load(1, 1)

    def body(kk, _):
        for b in range(2):
            l = 2 * kk + b
            wait(b)
            scat(l, b)

            @pl.when(l + 2 < NL)
            def _():
                load(l + 2, b)
        return 0

    lax.fori_loop(0, NL // 2, body, 0)

    plsc.subcore_barrier()

    pltpu.sync_copy(acc_sh.at[pl.ds(t * RPT, RPT)],
                    sums_hbm.at[c, pl.ds(t * RPT, RPT)])


@functools.partial(
    pl.kernel,
    mesh=_mesh,
    out_type=[
        jax.ShapeDtypeStruct((NC, NH, 16), jnp.float32),
    ],
    scratch_types=[
        pltpu.VMEM((NCH, C), jnp.int32),
        pltpu.VMEM((C, 16), jnp.float32),
        pltpu.VMEM((ZR, 16), jnp.float32),
        pltpu.VMEM_SHARED((NHP, 16), jnp.float32),
        pltpu.SemaphoreType.DMA,
    ],
)
def _sc_count(col3_hbm, counts_hbm, idx_v, ones_v, zero16_v, cnt_sh, sem):
    t = lax.axis_index("s")
    c = lax.axis_index("c")

    zeros16 = jnp.zeros((16,), jnp.float32)
    ones16 = jnp.ones((16,), jnp.float32)

    def zrow(i, _):
        zero16_v[i, :] = zeros16
        return 0

    lax.fori_loop(0, ZR, zrow, 0)

    def orow(i, _):
        ones_v[i, :] = ones16
        return 0

    lax.fori_loop(0, C, orow, 0)

    for j in range(RPT // ZR):
        pltpu.sync_copy(zero16_v, cnt_sh.at[pl.ds(t * RPT + j * ZR, ZR)])

    @pl.when(t == 0)
    def _zero_trash():
        pltpu.sync_copy(zero16_v.at[pl.ds(0, 8)], cnt_sh.at[pl.ds(NH, 8)])

    plsc.subcore_barrier()

    pltpu.sync_copy(col3_hbm.at[t], idx_v)
    _remap_indices(idx_v, c)

    WAVE = 10

    def body(kk, _):
        for j in range(WAVE):
            pltpu.async_copy(ones_v, cnt_sh.at[idx_v.at[kk * WAVE + j]], sem)
        for j in range(WAVE):
            pltpu.make_async_copy(ones_v, cnt_sh.at[idx_v.at[0]], sem).wait()
        return 0

    lax.fori_loop(0, NCH // WAVE, body, 0)

    plsc.subcore_barrier()

    pltpu.sync_copy(cnt_sh.at[pl.ds(t * RPT, RPT)],
                    counts_hbm.at[c, pl.ds(t * RPT, RPT)])


BN = 1000
_GRID = N // BN


def _mlp_body(sums_ref, counts_ref, node_ref, batch_ref, u_ref,
              w1_ref, b1_ref, w2_ref, b2_ref, out_ref):
    s = sums_ref[...]
    cnt = counts_ref[:, 0]
    agg = s / jnp.maximum(cnt, 1.0)[:, None]
    bvec = batch_ref[0, 0, :]
    onehot = (bvec[:, None]
              == lax.broadcasted_iota(jnp.int32, (BN, B), 1)
              ).astype(jnp.float32)
    ub = jnp.dot(onehot, u_ref[...],
                 preferred_element_type=jnp.float32)
    x = jnp.concatenate([node_ref[...], agg, ub], axis=1)
    h = jnp.dot(x, w1_ref[...], preferred_element_type=jnp.float32,
                precision=lax.Precision.HIGHEST) + b1_ref[...]
    h = jnp.maximum(h, 0.0)
    y = jnp.dot(h, w2_ref[...], preferred_element_type=jnp.float32,
                precision=lax.Precision.HIGHEST) + b2_ref[...]
    out_ref[...] = y


_mlp_call = pl.pallas_call(
    _mlp_body,
    grid=(_GRID,),
    in_specs=[
        pl.BlockSpec((BN, D), lambda i: (i, 0)),
        pl.BlockSpec((BN, 16), lambda i: (i, 0)),
        pl.BlockSpec((BN, D), lambda i: (i, 0)),
        pl.BlockSpec((1, 1, BN), lambda i: (i, 0, 0)),
        pl.BlockSpec((B, D), lambda i: (0, 0)),
        pl.BlockSpec((3 * D, H1), lambda i: (0, 0)),
        pl.BlockSpec((1, H1), lambda i: (0, 0)),
        pl.BlockSpec((H1, H2), lambda i: (0, 0)),
        pl.BlockSpec((1, H2), lambda i: (0, 0)),
    ],
    out_specs=pl.BlockSpec((BN, H2), lambda i: (i, 0)),
    out_shape=jax.ShapeDtypeStruct((N, H2), jnp.float32),
)


@jax.jit
def kernel(node_attr, edge_index, edge_attr, u, batch, W1, b1, W2, b2):
    col3 = edge_index[1].reshape(NS, NCH, C)
    (sums,) = _sc_scatter(edge_attr, col3)
    (counts,) = _sc_count(col3)
    sums_n = sums.reshape(NC * NH, D)
    counts_n = counts.reshape(NC * NH, 16)
    batch3 = batch.reshape(_GRID, 1, BN)
    return _mlp_call(sums_n, counts_n, node_attr, batch3, u,
                     W1, b1.reshape(1, H1), W2, b2.reshape(1, H2))


# speedup vs baseline: 3.5322x; 1.0748x over previous
import functools

import jax
import jax.numpy as jnp
from jax import lax
from jax.experimental import pallas as pl
from jax.experimental.pallas import tpu as pltpu
from jax.experimental.pallas import tpu_sc as plsc

N = 10000
E = 320000
D = 128
B = 64
H1 = 256
H2 = 128

NC = 2
NS = 16
EPT = E // NS
C = 80
NCH = EPT // C
CL = C
NL = EPT // CL
NH = 5120
TRASH = NH
NHP = NH + 8
RPT = NH // NS
ZR = 160

_mesh = plsc.VectorSubcoreMesh(core_axis_name="c", subcore_axis_name="s")


def _remap_indices(idx_v, c):
    lo = c * NH

    def remap(kk, _):
        for i in range(C // 16):
            iv = idx_v[kk, pl.ds(i * 16, 16)]
            local = iv - lo
            inrange = (local >= 0) & (local < NH)
            idx_v[kk, pl.ds(i * 16, 16)] = jnp.where(inrange, local, TRASH)
        return 0

    lax.fori_loop(0, NCH, remap, 0)


@functools.partial(
    pl.kernel,
    mesh=_mesh,
    out_type=[
        jax.ShapeDtypeStruct((NC, NH, D), jnp.float32),
    ],
    scratch_types=[
        pltpu.VMEM((NCH, C), jnp.int32),
        pltpu.VMEM((2, CL, D), jnp.float32),
        pltpu.VMEM((ZR, D), jnp.float32),
        pltpu.VMEM_SHARED((NHP, D), jnp.float32),
        pltpu.SemaphoreType.DMA,
        pltpu.SemaphoreType.DMA,
    ],
)
def _sc_scatter(edge_hbm, col3_hbm, sums_hbm, idx_v, rows_v, zero_v, acc_sh,
                sem0, sem1):
    t = lax.axis_index("s")
    c = lax.axis_index("c")

    zeros16 = jnp.zeros((16,), jnp.float32)

    def zrow(i, _):
        for j in range(D // 16):
            zero_v[i, pl.ds(j * 16, 16)] = zeros16
        return 0

    lax.fori_loop(0, ZR, zrow, 0)

    for j in range(RPT // ZR):
        pltpu.sync_copy(zero_v, acc_sh.at[pl.ds(t * RPT + j * ZR, ZR)])

    @pl.when(t == 0)
    def _zero_trash():
        pltpu.sync_copy(zero_v.at[pl.ds(0, 8)], acc_sh.at[pl.ds(NH, 8)])

    plsc.subcore_barrier()

    pltpu.sync_copy(col3_hbm.at[t], idx_v)
    _remap_indices(idx_v, c)

    ebase = t * EPT
    sems = (sem0, sem1)

    def load(l, buf):
        pltpu.async_copy(edge_hbm.at[pl.ds(ebase + l * CL, CL)],
                         rows_v.at[buf], sems[buf])

    def wait(buf):
        pltpu.make_async_copy(edge_hbm.at[pl.ds(0, CL)], rows_v.at[buf],
                              sems[buf]).wait()

    def scat(l, buf):
        pltpu.sync_copy(rows_v.at[buf], acc_sh.at[idx_v.at[l]], add=True)

    load(0, 0)
    load(1, 1)

    def body(kk, _):
        for b in range(2):
            l = 2 * kk + b
            wait(b)
            scat(l, b)

            @pl.when(l + 2 < NL)
            def _():
                load(l + 2, b)
        return 0

    lax.fori_loop(0, NL // 2, body, 0)

    plsc.subcore_barrier()

    pltpu.sync_copy(acc_sh.at[pl.ds(t * RPT, RPT)],
                    sums_hbm.at[c, pl.ds(t * RPT, RPT)])


@functools.partial(
    pl.kernel,
    mesh=_mesh,
    out_type=[
        jax.ShapeDtypeStruct((NC, NH, 16), jnp.float32),
    ],
    scratch_types=[
        pltpu.VMEM((NCH, C), jnp.int32),
        pltpu.VMEM((C, 16), jnp.float32),
        pltpu.VMEM((ZR, 16), jnp.float32),
        pltpu.VMEM_SHARED((NHP, 16), jnp.float32),
        pltpu.SemaphoreType.DMA,
    ],
)
def _sc_count(col3_hbm, counts_hbm, idx_v, ones_v, zero16_v, cnt_sh, sem):
    t = lax.axis_index("s")
    c = lax.axis_index("c")

    zeros16 = jnp.zeros((16,), jnp.float32)
    ones16 = jnp.ones((16,), jnp.float32)

    def zrow(i, _):
        zero16_v[i, :] = zeros16
        return 0

    lax.fori_loop(0, ZR, zrow, 0)

    def orow(i, _):
        ones_v[i, :] = ones16
        return 0

    lax.fori_loop(0, C, orow, 0)

    for j in range(RPT // ZR):
        pltpu.sync_copy(zero16_v, cnt_sh.at[pl.ds(t * RPT + j * ZR, ZR)])

    @pl.when(t == 0)
    def _zero_trash():
        pltpu.sync_copy(zero16_v.at[pl.ds(0, 8)], cnt_sh.at[pl.ds(NH, 8)])

    plsc.subcore_barrier()

    pltpu.sync_copy(col3_hbm.at[t], idx_v)
    _remap_indices(idx_v, c)

    WAVE = 25

    def body(kk, _):
        for j in range(WAVE):
            pltpu.async_copy(ones_v, cnt_sh.at[idx_v.at[kk * WAVE + j]], sem)
        for j in range(WAVE):
            pltpu.make_async_copy(ones_v, cnt_sh.at[idx_v.at[0]], sem).wait()
        return 0

    lax.fori_loop(0, NCH // WAVE, body, 0)

    plsc.subcore_barrier()

    pltpu.sync_copy(cnt_sh.at[pl.ds(t * RPT, RPT)],
                    counts_hbm.at[c, pl.ds(t * RPT, RPT)])


BN = 1000
_GRID = N // BN


def _mlp_body(sums_ref, counts_ref, node_ref, batch_ref, u_ref,
              w1_ref, b1_ref, w2_ref, b2_ref, out_ref):
    s = sums_ref[...]
    cnt = counts_ref[:, 0]
    agg = s / jnp.maximum(cnt, 1.0)[:, None]
    bvec = batch_ref[0, 0, :]
    onehot = (bvec[:, None]
              == lax.broadcasted_iota(jnp.int32, (BN, B), 1)
              ).astype(jnp.float32)
    ub = jnp.dot(onehot, u_ref[...],
                 preferred_element_type=jnp.float32)
    x = jnp.concatenate([node_ref[...], agg, ub], axis=1)
    h = jnp.dot(x, w1_ref[...],
                preferred_element_type=jnp.float32) + b1_ref[...]
    h = jnp.maximum(h, 0.0)
    y = jnp.dot(h, w2_ref[...],
                preferred_element_type=jnp.float32) + b2_ref[...]
    out_ref[...] = y


_mlp_call = pl.pallas_call(
    _mlp_body,
    grid=(_GRID,),
    in_specs=[
        pl.BlockSpec((BN, D), lambda i: (i, 0)),
        pl.BlockSpec((BN, 16), lambda i: (i, 0)),
        pl.BlockSpec((BN, D), lambda i: (i, 0)),
        pl.BlockSpec((1, 1, BN), lambda i: (i, 0, 0)),
        pl.BlockSpec((B, D), lambda i: (0, 0)),
        pl.BlockSpec((3 * D, H1), lambda i: (0, 0)),
        pl.BlockSpec((1, H1), lambda i: (0, 0)),
        pl.BlockSpec((H1, H2), lambda i: (0, 0)),
        pl.BlockSpec((1, H2), lambda i: (0, 0)),
    ],
    out_specs=pl.BlockSpec((BN, H2), lambda i: (i, 0)),
    out_shape=jax.ShapeDtypeStruct((N, H2), jnp.float32),
)


@jax.jit
def kernel(node_attr, edge_index, edge_attr, u, batch, W1, b1, W2, b2):
    col3 = edge_index[1].reshape(NS, NCH, C)
    (sums,) = _sc_scatter(edge_attr, col3)
    (counts,) = _sc_count(col3)
    sums_n = sums.reshape(NC * NH, D)
    counts_n = counts.reshape(NC * NH, 16)
    batch3 = batch.reshape(_GRID, 1, BN)
    return _mlp_call(sums_n, counts_n, node_attr, batch3, u,
                     W1, b1.reshape(1, H1), W2, b2.reshape(1, H2))


# speedup vs baseline: 3.5397x; 1.0021x over previous
import functools

import jax
import jax.numpy as jnp
from jax import lax
from jax.experimental import pallas as pl
from jax.experimental.pallas import tpu as pltpu
from jax.experimental.pallas import tpu_sc as plsc

N = 10000
E = 320000
D = 128
B = 64
H1 = 256
H2 = 128

NC = 2
NS = 16
EPT = E // NS
C = 80
NCH = EPT // C
CL = C
NL = EPT // CL
NH = 5120
TRASH = NH
NHP = NH + 8
RPT = NH // NS
ZR = 160

_mesh = plsc.VectorSubcoreMesh(core_axis_name="c", subcore_axis_name="s")


def _remap_indices(idx_v, c):
    lo = c * NH

    def remap(kk, _):
        for i in range(C // 16):
            iv = idx_v[kk, pl.ds(i * 16, 16)]
            local = iv - lo
            inrange = (local >= 0) & (local < NH)
            idx_v[kk, pl.ds(i * 16, 16)] = jnp.where(inrange, local, TRASH)
        return 0

    lax.fori_loop(0, NCH, remap, 0)


@functools.partial(
    pl.kernel,
    mesh=_mesh,
    out_type=[
        jax.ShapeDtypeStruct((NC, NH, D), jnp.float32),
    ],
    scratch_types=[
        pltpu.VMEM((NCH, C), jnp.int32),
        pltpu.VMEM((2, CL, D), jnp.float32),
        pltpu.VMEM((ZR, D), jnp.float32),
        pltpu.VMEM_SHARED((NHP, D), jnp.float32),
        pltpu.SemaphoreType.DMA,
        pltpu.SemaphoreType.DMA,
    ],
)
def _sc_scatter(edge_hbm, col3_hbm, sums_hbm, idx_v, rows_v, zero_v, acc_sh,
                sem0, sem1):
    t = lax.axis_index("s")
    c = lax.axis_index("c")

    zeros16 = jnp.zeros((16,), jnp.float32)

    def zrow(i, _):
        for j in range(D // 16):
            zero_v[i, pl.ds(j * 16, 16)] = zeros16
        return 0

    lax.fori_loop(0, ZR, zrow, 0)

    for j in range(RPT // ZR):
        pltpu.sync_copy(zero_v, acc_sh.at[pl.ds(t * RPT + j * ZR, ZR)])

    @pl.when(t == 0)
    def _zero_trash():
        pltpu.sync_copy(zero_v.at[pl.ds(0, 8)], acc_sh.at[pl.ds(NH, 8)])

    plsc.subcore_barrier()

    pltpu.sync_copy(col3_hbm.at[t], idx_v)
    _remap_indices(idx_v, c)

    ebase = t * EPT
    sems = (sem0, sem1)

    def load(l, buf):
        pltpu.async_copy(edge_hbm.at[pl.ds(ebase + l * CL, CL)],
                         rows_v.at[buf], sems[buf])

    def wait(buf):
        pltpu.make_async_copy(edge_hbm.at[pl.ds(0, CL)], rows_v.at[buf],
                              sems[buf]).wait()

    def scat(l, buf):
        pltpu.sync_copy(rows_v.at[buf], acc_sh.at[idx_v.at[l]], add=True)

    load(0, 0)
    load(1, 1)

    def body(kk, _):
        for b in range(2):
            l = 2 * kk + b
            wait(b)
            scat(l, b)

            @pl.when(l + 2 < NL)
            def _():
                load(l + 2, b)
        return 0

    lax.fori_loop(0, NL // 2, body, 0)

    plsc.subcore_barrier()

    pltpu.sync_copy(acc_sh.at[pl.ds(t * RPT, RPT)],
                    sums_hbm.at[c, pl.ds(t * RPT, RPT)])


@functools.partial(
    pl.kernel,
    mesh=_mesh,
    out_type=[
        jax.ShapeDtypeStruct((NC, NH, 16), jnp.float32),
    ],
    scratch_types=[
        pltpu.VMEM((NCH, C), jnp.int32),
        pltpu.VMEM((C, 16), jnp.float32),
        pltpu.VMEM((ZR, 16), jnp.float32),
        pltpu.VMEM_SHARED((NHP, 16), jnp.float32),
        pltpu.SemaphoreType.DMA,
    ],
)
def _sc_count(col3_hbm, counts_hbm, idx_v, ones_v, zero16_v, cnt_sh, sem):
    t = lax.axis_index("s")
    c = lax.axis_index("c")

    zeros16 = jnp.zeros((16,), jnp.float32)
    ones16 = jnp.ones((16,), jnp.float32)

    def zrow(i, _):
        zero16_v[i, :] = zeros16
        return 0

    lax.fori_loop(0, ZR, zrow, 0)

    def orow(i, _):
        ones_v[i, :] = ones16
        return 0

    lax.fori_loop(0, C, orow, 0)

    for j in range(RPT // ZR):
        pltpu.sync_copy(zero16_v, cnt_sh.at[pl.ds(t * RPT + j * ZR, ZR)])

    @pl.when(t == 0)
    def _zero_trash():
        pltpu.sync_copy(zero16_v.at[pl.ds(0, 8)], cnt_sh.at[pl.ds(NH, 8)])

    plsc.subcore_barrier()

    pltpu.sync_copy(col3_hbm.at[t], idx_v)
    _remap_indices(idx_v, c)

    def fire(k, _):
        pltpu.async_copy(ones_v, cnt_sh.at[idx_v.at[k]], sem)
        return 0

    lax.fori_loop(0, NCH, fire, 0)

    def drain(k, _):
        pltpu.make_async_copy(ones_v, cnt_sh.at[idx_v.at[0]], sem).wait()
        return 0

    lax.fori_loop(0, NCH, drain, 0)

    plsc.subcore_barrier()

    pltpu.sync_copy(cnt_sh.at[pl.ds(t * RPT, RPT)],
                    counts_hbm.at[c, pl.ds(t * RPT, RPT)])


BN = 1000
_GRID = N // BN


def _mlp_body(sums_ref, counts_ref, node_ref, batch_ref, u_ref,
              w1_ref, b1_ref, w2_ref, b2_ref, out_ref):
    s = sums_ref[...]
    cnt = counts_ref[:, 0]
    agg = s / jnp.maximum(cnt, 1.0)[:, None]
    bvec = batch_ref[0, 0, :]
    onehot = (bvec[:, None]
              == lax.broadcasted_iota(jnp.int32, (BN, B), 1)
              ).astype(jnp.float32)
    ub = jnp.dot(onehot, u_ref[...],
                 preferred_element_type=jnp.float32)
    x = jnp.concatenate([node_ref[...], agg, ub], axis=1)
    h = jnp.dot(x, w1_ref[...],
                preferred_element_type=jnp.float32) + b1_ref[...]
    h = jnp.maximum(h, 0.0)
    y = jnp.dot(h, w2_ref[...],
                preferred_element_type=jnp.float32) + b2_ref[...]
    out_ref[...] = y


_mlp_call = pl.pallas_call(
    _mlp_body,
    grid=(_GRID,),
    in_specs=[
        pl.BlockSpec((BN, D), lambda i: (i, 0)),
        pl.BlockSpec((BN, 16), lambda i: (i, 0)),
        pl.BlockSpec((BN, D), lambda i: (i, 0)),
        pl.BlockSpec((1, 1, BN), lambda i: (i, 0, 0)),
        pl.BlockSpec((B, D), lambda i: (0, 0)),
        pl.BlockSpec((3 * D, H1), lambda i: (0, 0)),
        pl.BlockSpec((1, H1), lambda i: (0, 0)),
        pl.BlockSpec((H1, H2), lambda i: (0, 0)),
        pl.BlockSpec((1, H2), lambda i: (0, 0)),
    ],
    out_specs=pl.BlockSpec((BN, H2), lambda i: (i, 0)),
    out_shape=jax.ShapeDtypeStruct((N, H2), jnp.float32),
)


@jax.jit
def kernel(node_attr, edge_index, edge_attr, u, batch, W1, b1, W2, b2):
    col3 = edge_index[1].reshape(NS, NCH, C)
    (sums,) = _sc_scatter(edge_attr, col3)
    (counts,) = _sc_count(col3)
    sums_n = sums.reshape(NC * NH, D)
    counts_n = counts.reshape(NC * NH, 16)
    batch3 = batch.reshape(_GRID, 1, BN)
    return _mlp_call(sums_n, counts_n, node_attr, batch3, u,
                     W1, b1.reshape(1, H1), W2, b2.reshape(1, H2))
